# baseline (speedup 1.0000x reference)
import functools

import jax
import jax.numpy as jnp
from jax import lax
from jax.experimental import pallas as pl
from jax.experimental.pallas import tpu as pltpu
from jax.experimental.pallas import tpu_sc as plsc

L = 16
NC = 2
NS = 16
N_BOX = 20
B_STRIDE = 128


def _make_sc_call(H, W, scale):
    rows_per = H // NS
    rch = H // L
    cch = W // L
    mesh = plsc.VectorSubcoreMesh(core_axis_name="c", subcore_axis_name="s")

    @functools.partial(
        pl.kernel,
        out_type=(
            jax.ShapeDtypeStruct((1, 1, H, W), jnp.float32),
            jax.ShapeDtypeStruct((1, 1, H, W), jnp.float32),
            jax.ShapeDtypeStruct((2 * L,), jnp.float32),
        ),
        mesh=mesh,
        compiler_params=pltpu.CompilerParams(needs_layout_passes=False),
        scratch_types=[
            pltpu.VMEM((2 * B_STRIDE,), jnp.float32),
            pltpu.VMEM((max(rch, cch) * L,), jnp.int32),
            pltpu.VMEM((H + L,), jnp.float32),
            pltpu.VMEM((W,), jnp.float32),
            pltpu.VMEM((rows_per, W), jnp.float32),
            pltpu.VMEM((L,), jnp.float32),
            pltpu.SemaphoreType.DMA,
        ],
    )
    def sc_body(bp_hbm, bc_hbm,
                mask_pre_hbm, mask_cur_hbm, norms_hbm,
                b_v, diff_v, rowm_v, colm_v, out_v, norms_v, sem):
        br = lax.axis_index("c")
        sid = lax.axis_index("s")
        rbase = sid * rows_per

        cp0 = pltpu.async_copy(bp_hbm, b_v.at[pl.ds(0, 5 * N_BOX)], sem)
        cp1 = pltpu.async_copy(bc_hbm, b_v.at[pl.ds(B_STRIDE, 5 * N_BOX)], sem)
        cp0.wait()
        cp1.wait()

        b_off = br * B_STRIDE
        iota = lax.iota(jnp.int32, L)
        i5 = iota * 5

        def build(lo_f, hi_f, n_chunks, dst_v):
            zero = jnp.zeros((L,), jnp.int32)

            def zbody(c, acc):
                diff_v[pl.ds(pl.multiple_of(c * L, L), L)] = zero
                return acc

            lax.fori_loop(0, n_chunks, zbody, 0)
            for h in range((N_BOX + L - 1) // L):
                base = b_off + h * 5 * L
                lane_ok = iota < (N_BOX - h * L)
                idx_lim = 2 * B_STRIDE - 1
                lo_i = jnp.minimum(i5 + (base + lo_f), idx_lim)
                hi_i = jnp.minimum(i5 + (base + hi_f), idx_lim)
                lab_i = jnp.minimum(i5 + (base + 4), idx_lim)
                lo = plsc.load_gather(b_v, [lo_i])
                hi = plsc.load_gather(b_v, [hi_i])
                lab = plsc.load_gather(b_v, [lab_i])
                cnd = jnp.where((lab != 0.0) & lane_ok,
                                jnp.int32(1), jnp.int32(0))
                loi = jnp.clip((lo * scale).astype(jnp.int32),
                               0, n_chunks * L - 1)
                hii = jnp.clip((hi * scale).astype(jnp.int32),
                               0, n_chunks * L - 1)
                plsc.addupdate_scatter(diff_v, [loi], cnd)
                plsc.addupdate_scatter(diff_v, [hii], -cnd)
            def cbody(c, carry_tot):
                carry, total = carry_tot
                sl = pl.ds(pl.multiple_of(c * L, L), L)
                dv = diff_v[sl]
                cs = plsc.cumsum(dv) + carry
                mi = cs > 0
                dst_v[sl] = mi.astype(jnp.float32)
                return (carry + jnp.broadcast_to(jnp.sum(dv), (L,)),
                        total + mi.astype(jnp.int32))

            _, total = lax.fori_loop(0, n_chunks, cbody, (zero, zero))
            return total

        rtot = build(1, 3, rch, rowm_v)
        ctot = build(0, 2, cch, colm_v)

        prod = (jnp.broadcast_to(jnp.sum(rtot), (L,))
                * jnp.broadcast_to(jnp.sum(ctot), (L,)) * 2)
        norms_v[pl.ds(0, L)] = jnp.where(
            prod > 0, prod.astype(jnp.float32), jnp.float32(1.0))

        zrow = jnp.zeros((L,), jnp.float32)
        myrows = rowm_v[pl.ds(rbase, L)]
        ons = [myrows[rr] > 0.0 for rr in range(rows_per)]

        def obody(c, acc):
            sl = pl.ds(pl.multiple_of(c * L, L), L)
            vec = colm_v[sl]
            for rr in range(rows_per):
                out_v[rr, sl] = jnp.where(ons[rr], vec, zrow)
            return acc

        lax.fori_loop(0, cch, obody, 0)

        @pl.when(br == 0)
        def _():
            pltpu.sync_copy(out_v,
                            mask_pre_hbm.at[0, 0, pl.ds(rbase, rows_per)])

            @pl.when(sid == 0)
            def _():
                pltpu.sync_copy(norms_v, norms_hbm.at[pl.ds(0, L)])

        @pl.when(br == 1)
        def _():
            pltpu.sync_copy(out_v,
                            mask_cur_hbm.at[0, 0, pl.ds(rbase, rows_per)])

            @pl.when(sid == 0)
            def _():
                pltpu.sync_copy(norms_v, norms_hbm.at[pl.ds(L, L)])

    return sc_body


def kernel(im_data, feature, gt_boxes_pre, num_boxes_pre, gt_boxes_cur,
           num_boxes_cur):
    H, W = feature.shape[2], feature.shape[3]
    H_img = im_data.shape[2]
    scale = float(H) / float(H_img)
    gp = (num_boxes_pre[0] > 0).astype(jnp.float32)
    gc = (num_boxes_cur[0] > 0).astype(jnp.float32)
    bp = (gt_boxes_pre * gp).reshape(-1)
    bc = (gt_boxes_cur * gc).reshape(-1)
    mask_pre, mask_cur, norms = _make_sc_call(H, W, scale)(bp, bc)
    return (mask_pre, norms[0], mask_cur, norms[L])


# speedup vs baseline: 1.0081x; 1.0081x over previous
import functools

import jax
import jax.numpy as jnp
from jax import lax
from jax.experimental import pallas as pl
from jax.experimental.pallas import tpu as pltpu
from jax.experimental.pallas import tpu_sc as plsc

L = 16
NC = 2
NS = 16
N_BOX = 20
B_STRIDE = 128


def _make_sc_call(H, W, scale):
    rows_per = H // NS
    rch = H // L
    cch = W // L
    mesh = plsc.VectorSubcoreMesh(core_axis_name="c", subcore_axis_name="s")

    @functools.partial(
        pl.kernel,
        out_type=(
            jax.ShapeDtypeStruct((1, 1, H, W), jnp.float32),
            jax.ShapeDtypeStruct((1, 1, H, W), jnp.float32),
            jax.ShapeDtypeStruct((2 * L,), jnp.float32),
        ),
        mesh=mesh,
        compiler_params=pltpu.CompilerParams(needs_layout_passes=False),
        scratch_types=[
            pltpu.VMEM((2 * B_STRIDE,), jnp.float32),
            pltpu.VMEM((max(rch, cch) * L,), jnp.int32),
            pltpu.VMEM((H + L,), jnp.float32),
            pltpu.VMEM((W,), jnp.float32),
            pltpu.VMEM((rows_per, W), jnp.float32),
            pltpu.VMEM((L,), jnp.float32),
            pltpu.SemaphoreType.DMA,
        ],
    )
    def sc_body(bp_hbm, bc_hbm,
                mask_pre_hbm, mask_cur_hbm, norms_hbm,
                b_v, diff_v, rowm_v, colm_v, out_v, norms_v, sem):
        br = lax.axis_index("c")
        sid = lax.axis_index("s")
        rbase = sid * rows_per

        cp0 = pltpu.async_copy(bp_hbm, b_v.at[pl.ds(0, 5 * N_BOX)], sem)
        cp1 = pltpu.async_copy(bc_hbm, b_v.at[pl.ds(B_STRIDE, 5 * N_BOX)], sem)
        cp0.wait()
        cp1.wait()

        b_off = br * B_STRIDE
        iota = lax.iota(jnp.int32, L)
        i5 = iota * 5

        def build(lo_f, hi_f, n_chunks, dst_v):
            zero = jnp.zeros((L,), jnp.int32)
            for c in range(n_chunks):
                diff_v[pl.ds(c * L, L)] = zero
            for h in range((N_BOX + L - 1) // L):
                base = b_off + h * 5 * L
                lane_ok = iota < (N_BOX - h * L)
                idx_lim = 2 * B_STRIDE - 1
                lo_i = jnp.minimum(i5 + (base + lo_f), idx_lim)
                hi_i = jnp.minimum(i5 + (base + hi_f), idx_lim)
                lab_i = jnp.minimum(i5 + (base + 4), idx_lim)
                lo = plsc.load_gather(b_v, [lo_i])
                hi = plsc.load_gather(b_v, [hi_i])
                lab = plsc.load_gather(b_v, [lab_i])
                cnd = jnp.where((lab != 0.0) & lane_ok,
                                jnp.int32(1), jnp.int32(0))
                loi = jnp.clip((lo * scale).astype(jnp.int32),
                               0, n_chunks * L - 1)
                hii = jnp.clip((hi * scale).astype(jnp.int32),
                               0, n_chunks * L - 1)
                plsc.addupdate_scatter(diff_v, [loi], cnd)
                plsc.addupdate_scatter(diff_v, [hii], -cnd)
            carry = zero
            total = zero
            for c in range(n_chunks):
                sl = pl.ds(c * L, L)
                dv = diff_v[sl]
                cs = plsc.cumsum(dv) + carry
                mi = cs > 0
                dst_v[sl] = mi.astype(jnp.float32)
                carry = carry + jnp.broadcast_to(jnp.sum(dv), (L,))
                total = total + mi.astype(jnp.int32)
            return total

        rtot = build(1, 3, rch, rowm_v)
        ctot = build(0, 2, cch, colm_v)

        prod = (jnp.broadcast_to(jnp.sum(rtot), (L,))
                * jnp.broadcast_to(jnp.sum(ctot), (L,)) * 2)
        norms_v[pl.ds(0, L)] = jnp.where(
            prod > 0, prod.astype(jnp.float32), jnp.float32(1.0))

        zrow = jnp.zeros((L,), jnp.float32)
        myrows = rowm_v[pl.ds(rbase, L)]
        ons = [myrows[rr] > 0.0 for rr in range(rows_per)]

        for c in range(cch):
            sl = pl.ds(c * L, L)
            vec = colm_v[sl]
            for rr in range(rows_per):
                out_v[rr, sl] = jnp.where(ons[rr], vec, zrow)

        @pl.when(br == 0)
        def _():
            pltpu.sync_copy(out_v,
                            mask_pre_hbm.at[0, 0, pl.ds(rbase, rows_per)])

            @pl.when(sid == 0)
            def _():
                pltpu.sync_copy(norms_v, norms_hbm.at[pl.ds(0, L)])

        @pl.when(br == 1)
        def _():
            pltpu.sync_copy(out_v,
                            mask_cur_hbm.at[0, 0, pl.ds(rbase, rows_per)])

            @pl.when(sid == 0)
            def _():
                pltpu.sync_copy(norms_v, norms_hbm.at[pl.ds(L, L)])

    return sc_body


def kernel(im_data, feature, gt_boxes_pre, num_boxes_pre, gt_boxes_cur,
           num_boxes_cur):
    H, W = feature.shape[2], feature.shape[3]
    H_img = im_data.shape[2]
    scale = float(H) / float(H_img)
    gp = (num_boxes_pre[0] > 0).astype(jnp.float32)
    gc = (num_boxes_cur[0] > 0).astype(jnp.float32)
    bp = (gt_boxes_pre * gp).reshape(-1)
    bc = (gt_boxes_cur * gc).reshape(-1)
    mask_pre, mask_cur, norms = _make_sc_call(H, W, scale)(bp, bc)
    return (mask_pre, norms[0], mask_cur, norms[L])


# speedup vs baseline: 1.0448x; 1.0364x over previous
import functools

import jax
import jax.numpy as jnp
from jax import lax
from jax.experimental import pallas as pl
from jax.experimental.pallas import tpu as pltpu
from jax.experimental.pallas import tpu_sc as plsc

L = 16
NC = 2
NS = 16
N_BOX = 20
B_STRIDE = 128


def _make_sc_call(H, W, scale):
    rows_per = H // NS
    rch = H // L
    cch = W // L
    mesh = plsc.VectorSubcoreMesh(core_axis_name="c", subcore_axis_name="s")

    @functools.partial(
        pl.kernel,
        out_type=(
            jax.ShapeDtypeStruct((1, 1, H, W), jnp.float32),
            jax.ShapeDtypeStruct((1, 1, H, W), jnp.float32),
            jax.ShapeDtypeStruct((1,), jnp.float32),
            jax.ShapeDtypeStruct((1,), jnp.float32),
        ),
        mesh=mesh,
        compiler_params=pltpu.CompilerParams(needs_layout_passes=False),
        scratch_types=[
            pltpu.VMEM((2 * B_STRIDE,), jnp.float32),
            pltpu.VMEM((max(rch, cch) * L,), jnp.int32),
            pltpu.VMEM((H + L,), jnp.float32),
            pltpu.VMEM((W,), jnp.float32),
            pltpu.VMEM((rows_per, W), jnp.float32),
            pltpu.VMEM((L,), jnp.float32),
            pltpu.SemaphoreType.DMA,
        ],
    )
    def sc_body(bp_hbm, bc_hbm,
                mask_pre_hbm, mask_cur_hbm, norms_pre_hbm, norms_cur_hbm,
                b_v, diff_v, rowm_v, colm_v, out_v, norms_v, sem):
        br = lax.axis_index("c")
        sid = lax.axis_index("s")
        rbase = sid * rows_per

        cp0 = pltpu.async_copy(bp_hbm, b_v.at[pl.ds(0, 5 * N_BOX)], sem)
        cp1 = pltpu.async_copy(bc_hbm, b_v.at[pl.ds(B_STRIDE, 5 * N_BOX)], sem)
        cp0.wait()
        cp1.wait()

        b_off = br * B_STRIDE
        iota = lax.iota(jnp.int32, L)
        i5 = iota * 5

        def build(lo_f, hi_f, n_chunks, dst_v):
            zero = jnp.zeros((L,), jnp.int32)
            for c in range(n_chunks):
                diff_v[pl.ds(c * L, L)] = zero
            for h in range((N_BOX + L - 1) // L):
                base = b_off + h * 5 * L
                lane_ok = iota < (N_BOX - h * L)
                idx_lim = 2 * B_STRIDE - 1
                lo_i = jnp.minimum(i5 + (base + lo_f), idx_lim)
                hi_i = jnp.minimum(i5 + (base + hi_f), idx_lim)
                lab_i = jnp.minimum(i5 + (base + 4), idx_lim)
                lo = plsc.load_gather(b_v, [lo_i])
                hi = plsc.load_gather(b_v, [hi_i])
                lab = plsc.load_gather(b_v, [lab_i])
                cnd = jnp.where((lab != 0.0) & lane_ok,
                                jnp.int32(1), jnp.int32(0))
                loi = jnp.clip((lo * scale).astype(jnp.int32),
                               0, n_chunks * L - 1)
                hii = jnp.clip((hi * scale).astype(jnp.int32),
                               0, n_chunks * L - 1)
                plsc.addupdate_scatter(diff_v, [loi], cnd)
                plsc.addupdate_scatter(diff_v, [hii], -cnd)
            carry = zero
            total = zero
            for c in range(n_chunks):
                sl = pl.ds(c * L, L)
                dv = diff_v[sl]
                cs = plsc.cumsum(dv) + carry
                mi = cs > 0
                dst_v[sl] = mi.astype(jnp.float32)
                carry = carry + jnp.broadcast_to(jnp.sum(dv), (L,))
                total = total + mi.astype(jnp.int32)
            return total

        rtot = build(1, 3, rch, rowm_v)
        ctot = build(0, 2, cch, colm_v)

        prod = (jnp.broadcast_to(jnp.sum(rtot), (L,))
                * jnp.broadcast_to(jnp.sum(ctot), (L,)) * 2)
        norms_v[pl.ds(0, L)] = jnp.where(
            prod > 0, prod.astype(jnp.float32), jnp.float32(1.0))

        zrow = jnp.zeros((L,), jnp.float32)
        myrows = rowm_v[pl.ds(rbase, L)]
        ons = [myrows[rr] > 0.0 for rr in range(rows_per)]

        for c in range(cch):
            sl = pl.ds(c * L, L)
            vec = colm_v[sl]
            for rr in range(rows_per):
                out_v[rr, sl] = jnp.where(ons[rr], vec, zrow)

        @pl.when(br == 0)
        def _():
            pltpu.sync_copy(out_v,
                            mask_pre_hbm.at[0, 0, pl.ds(rbase, rows_per)])

            @pl.when(sid == 0)
            def _():
                pltpu.sync_copy(norms_v.at[pl.ds(0, 1)], norms_pre_hbm)

        @pl.when(br == 1)
        def _():
            pltpu.sync_copy(out_v,
                            mask_cur_hbm.at[0, 0, pl.ds(rbase, rows_per)])

            @pl.when(sid == 0)
            def _():
                pltpu.sync_copy(norms_v.at[pl.ds(0, 1)], norms_cur_hbm)

    return sc_body


def kernel(im_data, feature, gt_boxes_pre, num_boxes_pre, gt_boxes_cur,
           num_boxes_cur):
    H, W = feature.shape[2], feature.shape[3]
    H_img = im_data.shape[2]
    scale = float(H) / float(H_img)
    gp = (num_boxes_pre[0] > 0).astype(jnp.float32)
    gc = (num_boxes_cur[0] > 0).astype(jnp.float32)
    bp = (gt_boxes_pre * gp).reshape(-1)
    bc = (gt_boxes_cur * gc).reshape(-1)
    mask_pre, mask_cur, norms_pre, norms_cur = _make_sc_call(H, W, scale)(
        bp, bc)
    return (mask_pre, norms_pre.reshape(()), mask_cur, norms_cur.reshape(()))
